# Initial kernel scaffold; baseline (speedup 1.0000x reference)
import jax
import jax.numpy as jnp
from jax.experimental import pallas as pl


def kernel(fs, ys, ft):
    raise NotImplementedError("write your pallas kernel here")


# speedup vs baseline: 3.9086x; 3.9086x over previous
import jax
import jax.numpy as jnp
from jax.experimental import pallas as pl
from jax.experimental.pallas import tpu as pltpu

_C = 10
_CP = 16
_B = 1024
_D = 4096
_BLK = 256
_NB = _B // _BLK
_GAMMA = 0.1
_EPS = 1e-8


def _body(fs_ref, ys_ref, ft_ref, out_ref, sums_ref, counts_ref):
    i = pl.program_id(0)

    @pl.when(i == 0)
    def _init():
        sums_ref[...] = jnp.zeros_like(sums_ref)
        counts_ref[...] = jnp.zeros_like(counts_ref)

    @pl.when(i < _NB)
    def _accum():
        ys = ys_ref[0, 0, :]
        classes = jax.lax.broadcasted_iota(jnp.int32, (_BLK, _CP), 1)
        onehot = (ys[:, None] == classes).astype(jnp.float32)
        fs = fs_ref[...]
        contrib = jax.lax.dot_general(
            onehot, fs, (((0,), (0,)), ((), ())),
            preferred_element_type=jnp.float32)
        sums_ref[...] += contrib
        counts_ref[...] += jnp.sum(onehot, axis=0, keepdims=True)

    @pl.when(i >= _NB)
    def _assign():
        ft = ft_ref[...]
        sums = sums_ref[...]
        counts = counts_ref[...]
        raw = jax.lax.dot_general(
            ft, sums, (((1,), (1,)), ((), ())),
            preferred_element_type=jnp.float32)
        scale = (1.0 - _GAMMA) * jnp.where(
            counts > 0.0, 1.0 / jnp.maximum(counts, 1.0), 0.0)
        sumsq = jnp.sum(sums * sums, axis=1)
        npr = scale * jnp.sqrt(sumsq).reshape(1, _CP)
        nf = jnp.sqrt(jnp.sum(ft * ft, axis=1, keepdims=True))
        denom = jnp.maximum(nf * npr, _EPS)
        cos = (raw * scale) / denom
        lane = jax.lax.broadcasted_iota(jnp.int32, (_BLK, _CP), 1)
        cos = jnp.where(lane < _C, cos, jnp.inf)
        labels = jnp.argmin(cos, axis=1).astype(jnp.int32)
        out_ref[...] = labels.reshape(1, 1, _BLK)


def kernel(fs, ys, ft):
    ys3 = ys.astype(jnp.int32).reshape(_NB, 1, _BLK)
    out = pl.pallas_call(
        _body,
        grid=(2 * _NB,),
        in_specs=[
            pl.BlockSpec((_BLK, _D), lambda i: (jnp.minimum(i, _NB - 1), 0)),
            pl.BlockSpec((1, 1, _BLK), lambda i: (jnp.minimum(i, _NB - 1), 0, 0)),
            pl.BlockSpec((_BLK, _D), lambda i: (jnp.maximum(i - _NB, 0), 0)),
        ],
        out_specs=pl.BlockSpec((1, 1, _BLK), lambda i: (jnp.maximum(i - _NB, 0), 0, 0)),
        out_shape=jax.ShapeDtypeStruct((_NB, 1, _BLK), jnp.int32),
        scratch_shapes=[
            pltpu.VMEM((_CP, _D), jnp.float32),
            pltpu.VMEM((1, _CP), jnp.float32),
        ],
        compiler_params=pltpu.CompilerParams(
            dimension_semantics=("arbitrary",)),
    )(fs, ys3, ft)
    return out.reshape(_B)


# speedup vs baseline: 3.9451x; 1.0093x over previous
import jax
import jax.numpy as jnp
from jax.experimental import pallas as pl
from jax.experimental.pallas import tpu as pltpu

_C = 10
_CP = 16
_B = 1024
_D = 4096
_BLK = 512
_NB = _B // _BLK
_GAMMA = 0.1
_EPS = 1e-8


def _body(fs_ref, ys_ref, ft_ref, out_ref, sums_ref, counts_ref):
    i = pl.program_id(0)

    @pl.when(i == 0)
    def _init():
        sums_ref[...] = jnp.zeros_like(sums_ref)
        counts_ref[...] = jnp.zeros_like(counts_ref)

    @pl.when(i < _NB)
    def _accum():
        ys = ys_ref[0, 0, :]
        classes = jax.lax.broadcasted_iota(jnp.int32, (_BLK, _CP), 1)
        onehot = (ys[:, None] == classes).astype(jnp.float32)
        fs = fs_ref[...]
        contrib = jax.lax.dot_general(
            onehot, fs, (((0,), (0,)), ((), ())),
            preferred_element_type=jnp.float32)
        sums_ref[...] += contrib
        counts_ref[...] += jnp.sum(onehot, axis=0, keepdims=True)

    @pl.when(i >= _NB)
    def _assign():
        ft = ft_ref[...]
        sums = sums_ref[...]
        counts = counts_ref[...]
        raw = jax.lax.dot_general(
            ft, sums, (((1,), (1,)), ((), ())),
            preferred_element_type=jnp.float32)
        scale = (1.0 - _GAMMA) * jnp.where(
            counts > 0.0, 1.0 / jnp.maximum(counts, 1.0), 0.0)
        sumsq = jnp.sum(sums * sums, axis=1)
        npr = scale * jnp.sqrt(sumsq).reshape(1, _CP)
        nf = jnp.sqrt(jnp.sum(ft * ft, axis=1, keepdims=True))
        denom = jnp.maximum(nf * npr, _EPS)
        cos = (raw * scale) / denom
        lane = jax.lax.broadcasted_iota(jnp.int32, (_BLK, _CP), 1)
        cos = jnp.where(lane < _C, cos, jnp.inf)
        labels = jnp.argmin(cos, axis=1).astype(jnp.int32)
        out_ref[...] = labels.reshape(1, 1, _BLK)


def kernel(fs, ys, ft):
    ys3 = ys.astype(jnp.int32).reshape(_NB, 1, _BLK)
    out = pl.pallas_call(
        _body,
        grid=(2 * _NB,),
        in_specs=[
            pl.BlockSpec((_BLK, _D), lambda i: (jnp.minimum(i, _NB - 1), 0)),
            pl.BlockSpec((1, 1, _BLK), lambda i: (jnp.minimum(i, _NB - 1), 0, 0)),
            pl.BlockSpec((_BLK, _D), lambda i: (jnp.maximum(i - _NB, 0), 0)),
        ],
        out_specs=pl.BlockSpec((1, 1, _BLK), lambda i: (jnp.maximum(i - _NB, 0), 0, 0)),
        out_shape=jax.ShapeDtypeStruct((_NB, 1, _BLK), jnp.int32),
        scratch_shapes=[
            pltpu.VMEM((_CP, _D), jnp.float32),
            pltpu.VMEM((1, _CP), jnp.float32),
        ],
        compiler_params=pltpu.CompilerParams(
            dimension_semantics=("arbitrary",)),
    )(fs, ys3, ft)
    return out.reshape(_B)
